# Initial kernel scaffold; baseline (speedup 1.0000x reference)
import jax
import jax.numpy as jnp
from jax.experimental import pallas as pl


def kernel(constraint_features, edge_indices, edge_features, variable_features, params):
    raise NotImplementedError("write your pallas kernel here")


# speedup vs baseline: 1.8903x; 1.8903x over previous
import functools

import jax
import jax.numpy as jnp
from jax import lax
from jax.experimental import pallas as pl
from jax.experimental.pallas import tpu as pltpu
from jax.experimental.pallas import tpu_sc as plsc

N_NODES = 50000
N_EDGES = 800000
D = 64
H = 32
LANES = 16
NTILES = 16
CHUNK = 128
NCHUNKS = N_EDGES // CHUNK
ITERS = (NCHUNKS + NTILES - 1) // NTILES
RCHUNK = 128
NRFULL = N_NODES // RCHUNK
RTAIL = N_NODES - NRFULL * RCHUNK
RITERS = (NRFULL + 1 + NTILES - 1) // NTILES


def _tanh16(x):
    e = jnp.exp(jnp.abs(x) * -2.0)
    q = (1.0 - e) / (1.0 + e)
    return jnp.where(x < 0.0, -q, q)


def _sc_deg_body(eidx_hbm, out_hbm, t_sh, idx_v, ones_v, zbuf):
    c = lax.axis_index("c")
    s = lax.axis_index("s")

    def fill_rows(i, _):
        zbuf[i, pl.ds(0, LANES)] = jnp.zeros((LANES,), jnp.float32)
        ones_v[i, pl.ds(0, LANES)] = jnp.full((LANES,), 1.0, jnp.float32)
        return 0
    lax.fori_loop(0, CHUNK, fill_rows, 0)

    def zstep(j, _):
        cid = j * NTILES + s
        off = cid * RCHUNK

        @pl.when(cid < NRFULL)
        def _():
            pltpu.sync_copy(zbuf, t_sh.at[pl.ds(off, RCHUNK)])

        @pl.when(cid == NRFULL)
        def _():
            pltpu.sync_copy(zbuf.at[pl.ds(0, RTAIL)], t_sh.at[pl.ds(off, RTAIL)])
        return 0
    lax.fori_loop(0, RITERS, zstep, 0)
    plsc.subcore_barrier()

    def step(j, _):
        chunk_id = j * NTILES + s

        @pl.when(chunk_id < NCHUNKS)
        def _():
            off = chunk_id * CHUNK
            pltpu.sync_copy(eidx_hbm.at[c, pl.ds(off, CHUNK)], idx_v)
            pltpu.sync_copy(ones_v, t_sh.at[idx_v], add=True)
        return 0
    lax.fori_loop(0, ITERS, step, 0)
    plsc.subcore_barrier()

    def fstep(j, _):
        cid = j * NTILES + s
        off = cid * RCHUNK

        @pl.when(cid < NRFULL)
        def _():
            pltpu.sync_copy(t_sh.at[pl.ds(off, RCHUNK)],
                            out_hbm.at[c, pl.ds(off, RCHUNK)])

        @pl.when(cid == NRFULL)
        def _():
            pltpu.sync_copy(t_sh.at[pl.ds(off, RTAIL)],
                            out_hbm.at[c, pl.ds(off, RTAIL)])
        return 0
    lax.fori_loop(0, RITERS, fstep, 0)


def _sc_deg(eidx):
    k = pl.kernel(
        _sc_deg_body,
        out_type=jax.ShapeDtypeStruct((2, N_NODES, LANES), jnp.float32),
        mesh=plsc.VectorSubcoreMesh(core_axis_name="c", subcore_axis_name="s"),
        scratch_types=[
            pltpu.VMEM_SHARED((N_NODES, LANES), jnp.float32),
            pltpu.VMEM((CHUNK,), jnp.int32),
            pltpu.VMEM((CHUNK, LANES), jnp.float32),
            pltpu.VMEM((CHUNK, LANES), jnp.float32),
        ],
        compiler_params=pltpu.CompilerParams(use_tc_tiling_on_sc=False),
    )
    return k(eidx)


def _sc_conv_body(a_hbm, b_hbm, dst_hbm, src_hbm, ew_hbm, fe_hbm, out_hbm,
                  t_sh, idx_d, idx_s, idx_da, idx_sa, ew_v,
                  arows, brows, payload, fe_v, zbuf, sem_a, sem_b):
    c = lax.axis_index("c")
    s = lax.axis_index("s")

    def zrow(i, _):
        zbuf[i, pl.ds(0, LANES)] = jnp.zeros((LANES,), jnp.float32)
        zbuf[i, pl.ds(LANES, LANES)] = jnp.zeros((LANES,), jnp.float32)
        return 0
    lax.fori_loop(0, CHUNK, zrow, 0)

    def zstep(j, _):
        cid = j * NTILES + s
        off = cid * RCHUNK

        @pl.when(cid < NRFULL)
        def _():
            pltpu.sync_copy(zbuf, t_sh.at[pl.ds(off, RCHUNK)])

        @pl.when(cid == NRFULL)
        def _():
            pltpu.sync_copy(zbuf.at[pl.ds(0, RTAIL)], t_sh.at[pl.ds(off, RTAIL)])
        return 0
    lax.fori_loop(0, RITERS, zstep, 0)

    pltpu.sync_copy(fe_hbm.at[c], fe_v)
    fe0 = fe_v[pl.ds(0, LANES)]
    fe1 = fe_v[pl.ds(LANES, LANES)]
    coff = c * N_NODES
    plsc.subcore_barrier()

    def step(j, _):
        chunk_id = j * NTILES + s

        @pl.when(chunk_id < NCHUNKS)
        def _():
            off = chunk_id * CHUNK
            pltpu.sync_copy(dst_hbm.at[pl.ds(off, CHUNK)], idx_d)
            pltpu.sync_copy(src_hbm.at[pl.ds(off, CHUNK)], idx_s)
            pltpu.sync_copy(ew_hbm.at[pl.ds(off, CHUNK)], ew_v.at[pl.ds(0, CHUNK)])

            def adj(k, _):
                sl = pl.ds(k * LANES, LANES)
                idx_da[sl] = idx_d[sl] + coff
                idx_sa[sl] = idx_s[sl] + coff
                return 0
            lax.fori_loop(0, CHUNK // LANES, adj, 0)

            cp_a = pltpu.async_copy(a_hbm.at[idx_da], arows, sem_a)
            cp_b = pltpu.async_copy(b_hbm.at[idx_sa], brows, sem_b)
            cp_a.wait()
            cp_b.wait()

            def edge(i, _):
                ew16 = jnp.full((LANES,), ew_v[pl.ds(i, LANES)][0], jnp.float32)
                a0 = arows[i, pl.ds(0, LANES)]
                b0 = brows[i, pl.ds(0, LANES)]
                payload[i, pl.ds(0, LANES)] = _tanh16(a0 + b0 + ew16 * fe0)
                a1 = arows[i, pl.ds(LANES, LANES)]
                b1 = brows[i, pl.ds(LANES, LANES)]
                payload[i, pl.ds(LANES, LANES)] = _tanh16(a1 + b1 + ew16 * fe1)
                return 0
            lax.fori_loop(0, CHUNK, edge, 0)

            pltpu.sync_copy(payload, t_sh.at[idx_d], add=True)
        return 0
    lax.fori_loop(0, ITERS, step, 0)
    plsc.subcore_barrier()

    def fstep(j, _):
        cid = j * NTILES + s
        off = cid * RCHUNK

        @pl.when(cid < NRFULL)
        def _():
            pltpu.sync_copy(t_sh.at[pl.ds(off, RCHUNK)],
                            out_hbm.at[c, pl.ds(off, RCHUNK)])

        @pl.when(cid == NRFULL)
        def _():
            pltpu.sync_copy(t_sh.at[pl.ds(off, RTAIL)],
                            out_hbm.at[c, pl.ds(off, RTAIL)])
        return 0
    lax.fori_loop(0, RITERS, fstep, 0)


def _sc_conv(a_cat, b_cat, dst, src, ew, fe_cat):
    k = pl.kernel(
        _sc_conv_body,
        out_type=jax.ShapeDtypeStruct((2, N_NODES, H), jnp.float32),
        mesh=plsc.VectorSubcoreMesh(core_axis_name="c", subcore_axis_name="s"),
        scratch_types=[
            pltpu.VMEM_SHARED((N_NODES, H), jnp.float32),
            pltpu.VMEM((CHUNK,), jnp.int32),
            pltpu.VMEM((CHUNK,), jnp.int32),
            pltpu.VMEM((CHUNK,), jnp.int32),
            pltpu.VMEM((CHUNK,), jnp.int32),
            pltpu.VMEM((CHUNK + LANES,), jnp.float32),
            pltpu.VMEM((CHUNK, H), jnp.float32),
            pltpu.VMEM((CHUNK, H), jnp.float32),
            pltpu.VMEM((CHUNK, H), jnp.float32),
            pltpu.VMEM((H,), jnp.float32),
            pltpu.VMEM((CHUNK, H), jnp.float32),
            pltpu.SemaphoreType.DMA,
            pltpu.SemaphoreType.DMA,
        ],
        compiler_params=pltpu.CompilerParams(use_tc_tiling_on_sc=False),
    )
    return k(a_cat, b_cat, dst, src, ew, fe_cat)


BM = 2000


def _full_spec(shape):
    nd = len(shape)
    return pl.BlockSpec(shape, lambda i: (0,) * nd)


def _rows_spec(width):
    return pl.BlockSpec((BM, width), lambda i: (i, 0))


def _dot(x, w):
    return jnp.dot(x, w, preferred_element_type=jnp.float32)


def _tc0_body(cf, vf, wc1, bc1, wc2, bc2, wv1, bv1, wv2, bv2,
              fl1, flb1, fr1, fl2, flb2,
              c0_o, v0_o, a1_o, b1_o, a2_o):
    c0 = jnp.tanh(_dot(jnp.tanh(_dot(cf[...], wc1[...]) + bc1[...]), wc2[...]) + bc2[...])
    v0 = jnp.tanh(_dot(jnp.tanh(_dot(vf[...], wv1[...]) + bv1[...]), wv2[...]) + bv2[...])
    c0_o[...] = c0
    v0_o[...] = v0
    a1_o[...] = _dot(c0, fl1[...]) + flb1[...]
    b1_o[...] = _dot(v0, fr1[...])
    a2_o[...] = _dot(v0, fl2[...]) + flb2[...]


def _tc_mid_body(t1, degc, c0, ffw, ffb, o1a, o1b, o1bias, o2w, o2b, fr2,
                 b2_o):
    agg = _dot(t1[...], ffw[...]) + degc[...][:, 0:1] * ffb[...]
    h = jnp.tanh(_dot(agg, o1a[...]) + _dot(c0[...], o1b[...]) + o1bias[...])
    c1 = _dot(h, o2w[...]) + o2b[...]
    b2_o[...] = _dot(c1, fr2[...])


def _tc_fin_body(t2, degv, v0, ffw, ffb, o1a, o1b, o1bias, o2w, o2b,
                 w1, b1, w2, out_o):
    agg = _dot(t2[...], ffw[...]) + degv[...][:, 0:1] * ffb[...]
    h = jnp.tanh(_dot(agg, o1a[...]) + _dot(v0[...], o1b[...]) + o1bias[...])
    v1 = _dot(h, o2w[...]) + o2b[...]
    out_o[...] = _dot(jnp.tanh(_dot(v1, w1[...]) + b1[...]), w2[...])


def _tc_call(body, ins, widths_in, outs_shapes):
    grid = (N_NODES // BM,)
    in_specs = []
    for x, w in zip(ins, widths_in):
        in_specs.append(_rows_spec(w) if w is not None else _full_spec(x.shape))
    return pl.pallas_call(
        body,
        grid=grid,
        in_specs=in_specs,
        out_specs=[_rows_spec(s[1]) for s in outs_shapes],
        out_shape=[jax.ShapeDtypeStruct(s, jnp.float32) for s in outs_shapes],
    )(*ins)


def _split_halves(x):
    return jnp.concatenate([x[:, :H], x[:, H:]], axis=0)


def kernel(constraint_features, edge_indices, edge_features, variable_features, params):
    p = params
    eidx = edge_indices.astype(jnp.int32)
    ew = edge_features[:, 0]
    vc, cv = p["vc"], p["cv"]

    def r1(b):
        return b.reshape(1, -1)

    c0, v0, a1, b1, a2 = _tc_call(
        _tc0_body,
        [constraint_features, variable_features,
         p["c_emb"]["W1"], r1(p["c_emb"]["b1"]), p["c_emb"]["W2"], r1(p["c_emb"]["b2"]),
         p["v_emb"]["W1"], r1(p["v_emb"]["b1"]), p["v_emb"]["W2"], r1(p["v_emb"]["b2"]),
         vc["fl_W"], r1(vc["fl_b"]), vc["fr_W"], cv["fl_W"], r1(cv["fl_b"])],
        [5, 17] + [None] * 13,
        [(N_NODES, D)] * 5,
    )

    degs = _sc_deg(eidx)

    t1 = _sc_conv(_split_halves(a1), _split_halves(b1),
                  eidx[0], eidx[1], ew, vc["fe_W"][0].reshape(2, H))
    t1_full = jnp.concatenate([t1[0], t1[1]], axis=1)

    (b2,) = _tc_call(
        _tc_mid_body,
        [t1_full, degs[0], c0,
         vc["ff_W"], r1(vc["ff_b"]), vc["o1_W"][:D], vc["o1_W"][D:],
         r1(vc["o1_b"]), vc["o2_W"], r1(vc["o2_b"]), cv["fr_W"]],
        [D, LANES, D] + [None] * 8,
        [(N_NODES, D)],
    )

    t2 = _sc_conv(_split_halves(a2), _split_halves(b2),
                  eidx[1], eidx[0], ew, cv["fe_W"][0].reshape(2, H))
    t2_full = jnp.concatenate([t2[0], t2[1]], axis=1)

    (out,) = _tc_call(
        _tc_fin_body,
        [t2_full, degs[1], v0,
         cv["ff_W"], r1(cv["ff_b"]), cv["o1_W"][:D], cv["o1_W"][D:],
         r1(cv["o1_b"]), cv["o2_W"], r1(cv["o2_b"]),
         p["out"]["W1"], r1(p["out"]["b1"]), p["out"]["W2"]],
        [D, LANES, D] + [None] * 10,
        [(N_NODES, 1)],
    )
    return out


# speedup vs baseline: 2.5627x; 1.3557x over previous
import functools

import jax
import jax.numpy as jnp
from jax import lax
from jax.experimental import pallas as pl
from jax.experimental.pallas import tpu as pltpu
from jax.experimental.pallas import tpu_sc as plsc

N_NODES = 50000
N_EDGES = 800000
D = 64
H = 32
LANES = 16
NTILES = 16
CHUNK = 128
NCHUNKS = N_EDGES // CHUNK
ITERS = (NCHUNKS + NTILES - 1) // NTILES
RCHUNK = 128
NRFULL = N_NODES // RCHUNK
RTAIL = N_NODES - NRFULL * RCHUNK
RITERS = (NRFULL + 1 + NTILES - 1) // NTILES


def _tanh16(x):
    e = jnp.exp(jnp.abs(x) * -2.0)
    q = (1.0 - e) / (1.0 + e)
    return jnp.where(x < 0.0, -q, q)


def _sc_deg_body(eidx_hbm, out_hbm, t_sh, idx0, idx1, ones_v, zbuf,
                 sem_l0, sem_l1):
    c = lax.axis_index("c")
    s = lax.axis_index("s")
    idx_v = (idx0, idx1)
    sem_l = (sem_l0, sem_l1)

    def fill_rows(i, _):
        zbuf[i, pl.ds(0, LANES)] = jnp.zeros((LANES,), jnp.float32)
        ones_v[i, pl.ds(0, LANES)] = jnp.full((LANES,), 1.0, jnp.float32)
        return 0
    lax.fori_loop(0, CHUNK, fill_rows, 0)

    def lin(j, b):
        off = (j * NTILES + s) * CHUNK
        return pltpu.make_async_copy(eidx_hbm.at[c, pl.ds(off, CHUNK)],
                                     idx_v[b], sem_l[b])

    def valid(j):
        return (j * NTILES + s) < NCHUNKS

    lin(0, 0).start()
    lin(1, 1).start()

    def zstep(j, _):
        cid = j * NTILES + s
        off = cid * RCHUNK

        @pl.when(cid < NRFULL)
        def _():
            pltpu.sync_copy(zbuf, t_sh.at[pl.ds(off, RCHUNK)])

        @pl.when(cid == NRFULL)
        def _():
            pltpu.sync_copy(zbuf.at[pl.ds(0, RTAIL)], t_sh.at[pl.ds(off, RTAIL)])
        return 0
    lax.fori_loop(0, RITERS, zstep, 0)
    plsc.subcore_barrier()

    def step(jj, _):
        for b in (0, 1):
            j = jj * 2 + b

            @pl.when(valid(j))
            def _():
                lin(j, b).wait()
                pltpu.sync_copy(ones_v, t_sh.at[idx_v[b]], add=True)

            @pl.when(valid(j + 2))
            def _():
                lin(j + 2, b).start()
        return 0
    lax.fori_loop(0, (ITERS + 1) // 2, step, 0)
    plsc.subcore_barrier()

    def fstep(j, _):
        cid = j * NTILES + s
        off = cid * RCHUNK

        @pl.when(cid < NRFULL)
        def _():
            pltpu.sync_copy(t_sh.at[pl.ds(off, RCHUNK)],
                            out_hbm.at[c, pl.ds(off, RCHUNK)])

        @pl.when(cid == NRFULL)
        def _():
            pltpu.sync_copy(t_sh.at[pl.ds(off, RTAIL)],
                            out_hbm.at[c, pl.ds(off, RTAIL)])
        return 0
    lax.fori_loop(0, RITERS, fstep, 0)


def _sc_deg(eidx):
    k = pl.kernel(
        _sc_deg_body,
        out_type=jax.ShapeDtypeStruct((2, N_NODES, LANES), jnp.float32),
        mesh=plsc.VectorSubcoreMesh(core_axis_name="c", subcore_axis_name="s"),
        scratch_types=[
            pltpu.VMEM_SHARED((N_NODES, LANES), jnp.float32),
            pltpu.VMEM((CHUNK,), jnp.int32),
            pltpu.VMEM((CHUNK,), jnp.int32),
            pltpu.VMEM((CHUNK, LANES), jnp.float32),
            pltpu.VMEM((CHUNK, LANES), jnp.float32),
            pltpu.SemaphoreType.DMA,
            pltpu.SemaphoreType.DMA,
        ],
        compiler_params=pltpu.CompilerParams(use_tc_tiling_on_sc=False),
    )
    return k(eidx)


def _sc_conv_body(a_hbm, b_hbm, dst_hbm, src_hbm, ew_hbm, fe_hbm, out_hbm,
                  t_sh,
                  idx_d0, idx_d1, idx_s0, idx_s1, idx_da0, idx_da1,
                  idx_sa0, idx_sa1, ew_v0, ew_v1, arows0, arows1,
                  brows0, brows1, payload, fe_v, zbuf,
                  sem_l0, sem_l1, sem_g0, sem_g1):
    c = lax.axis_index("c")
    s = lax.axis_index("s")
    idx_d = (idx_d0, idx_d1)
    idx_s = (idx_s0, idx_s1)
    idx_da = (idx_da0, idx_da1)
    idx_sa = (idx_sa0, idx_sa1)
    ew_v = (ew_v0, ew_v1)
    arows = (arows0, arows1)
    brows = (brows0, brows1)
    sem_l = (sem_l0, sem_l1)
    sem_g = (sem_g0, sem_g1)

    def valid(j):
        return (j * NTILES + s) < NCHUNKS

    def lin_descs(j, b):
        off = (j * NTILES + s) * CHUNK
        return (
            pltpu.make_async_copy(dst_hbm.at[pl.ds(off, CHUNK)], idx_d[b], sem_l[b]),
            pltpu.make_async_copy(src_hbm.at[pl.ds(off, CHUNK)], idx_s[b], sem_l[b]),
            pltpu.make_async_copy(ew_hbm.at[pl.ds(off, CHUNK)],
                                  ew_v[b].at[pl.ds(0, CHUNK)], sem_l[b]),
        )

    def gat_descs(b):
        return (
            pltpu.make_async_copy(a_hbm.at[idx_da[b]], arows[b], sem_g[b]),
            pltpu.make_async_copy(b_hbm.at[idx_sa[b]], brows[b], sem_g[b]),
        )

    coff = c * N_NODES

    def adj(b):
        def body(k, _):
            sl = pl.ds(k * LANES, LANES)
            idx_da[b][sl] = idx_d[b][sl] + coff
            idx_sa[b][sl] = idx_s[b][sl] + coff
            return 0
        lax.fori_loop(0, CHUNK // LANES, body, 0)

    for d in lin_descs(0, 0):
        d.start()
    for d in lin_descs(1, 1):
        d.start()

    def zrow(i, _):
        zbuf[i, pl.ds(0, LANES)] = jnp.zeros((LANES,), jnp.float32)
        zbuf[i, pl.ds(LANES, LANES)] = jnp.zeros((LANES,), jnp.float32)
        return 0
    lax.fori_loop(0, CHUNK, zrow, 0)

    def zstep(j, _):
        cid = j * NTILES + s
        off = cid * RCHUNK

        @pl.when(cid < NRFULL)
        def _():
            pltpu.sync_copy(zbuf, t_sh.at[pl.ds(off, RCHUNK)])

        @pl.when(cid == NRFULL)
        def _():
            pltpu.sync_copy(zbuf.at[pl.ds(0, RTAIL)], t_sh.at[pl.ds(off, RTAIL)])
        return 0
    lax.fori_loop(0, RITERS, zstep, 0)

    pltpu.sync_copy(fe_hbm.at[c], fe_v)
    fe0 = fe_v[pl.ds(0, LANES)]
    fe1 = fe_v[pl.ds(LANES, LANES)]

    for d in lin_descs(0, 0):
        d.wait()
    adj(0)
    for d in gat_descs(0):
        d.start()
    plsc.subcore_barrier()

    def step(jj, _):
        for b in (0, 1):
            j = jj * 2 + b
            nb = 1 - b

            @pl.when(valid(j + 1))
            def _():
                for d in lin_descs(j + 1, nb):
                    d.wait()
                adj(nb)
                for d in gat_descs(nb):
                    d.start()

            @pl.when(valid(j))
            def _():
                for d in gat_descs(b):
                    d.wait()

                def edge(i, _):
                    ew16 = jnp.full((LANES,), ew_v[b][pl.ds(i, LANES)][0],
                                    jnp.float32)
                    a0 = arows[b][i, pl.ds(0, LANES)]
                    b0 = brows[b][i, pl.ds(0, LANES)]
                    payload[i, pl.ds(0, LANES)] = _tanh16(a0 + b0 + ew16 * fe0)
                    a1 = arows[b][i, pl.ds(LANES, LANES)]
                    b1 = brows[b][i, pl.ds(LANES, LANES)]
                    payload[i, pl.ds(LANES, LANES)] = _tanh16(a1 + b1 + ew16 * fe1)
                    return 0
                lax.fori_loop(0, CHUNK, edge, 0)

                pltpu.sync_copy(payload, t_sh.at[idx_d[b]], add=True)

            @pl.when(valid(j + 2))
            def _():
                for d in lin_descs(j + 2, b):
                    d.start()
        return 0
    lax.fori_loop(0, (ITERS + 1) // 2, step, 0)
    plsc.subcore_barrier()

    def fstep(j, _):
        cid = j * NTILES + s
        off = cid * RCHUNK

        @pl.when(cid < NRFULL)
        def _():
            pltpu.sync_copy(t_sh.at[pl.ds(off, RCHUNK)],
                            out_hbm.at[c, pl.ds(off, RCHUNK)])

        @pl.when(cid == NRFULL)
        def _():
            pltpu.sync_copy(t_sh.at[pl.ds(off, RTAIL)],
                            out_hbm.at[c, pl.ds(off, RTAIL)])
        return 0
    lax.fori_loop(0, RITERS, fstep, 0)


def _sc_conv(a_cat, b_cat, dst, src, ew, fe_cat):
    k = pl.kernel(
        _sc_conv_body,
        out_type=jax.ShapeDtypeStruct((2, N_NODES, H), jnp.float32),
        mesh=plsc.VectorSubcoreMesh(core_axis_name="c", subcore_axis_name="s"),
        scratch_types=(
            [pltpu.VMEM_SHARED((N_NODES, H), jnp.float32)]
            + [pltpu.VMEM((CHUNK,), jnp.int32)] * 8
            + [pltpu.VMEM((CHUNK + LANES,), jnp.float32)] * 2
            + [pltpu.VMEM((CHUNK, H), jnp.float32)] * 5
            + [pltpu.VMEM((H,), jnp.float32)]
            + [pltpu.VMEM((CHUNK, H), jnp.float32)]
            + [pltpu.SemaphoreType.DMA] * 4
        ),
        compiler_params=pltpu.CompilerParams(use_tc_tiling_on_sc=False),
    )
    return k(a_cat, b_cat, dst, src, ew, fe_cat)


BM = 2000


def _full_spec(shape):
    nd = len(shape)
    return pl.BlockSpec(shape, lambda i: (0,) * nd)


def _rows_spec(width):
    return pl.BlockSpec((BM, width), lambda i: (i, 0))


def _dot(x, w):
    return jnp.dot(x, w, preferred_element_type=jnp.float32)


def _tc0_body(cf, vf, wc1, bc1, wc2, bc2, wv1, bv1, wv2, bv2,
              fl1, flb1, fr1, fl2, flb2,
              c0_o, v0_o, a1_o, b1_o, a2_o):
    c0 = jnp.tanh(_dot(jnp.tanh(_dot(cf[...], wc1[...]) + bc1[...]), wc2[...]) + bc2[...])
    v0 = jnp.tanh(_dot(jnp.tanh(_dot(vf[...], wv1[...]) + bv1[...]), wv2[...]) + bv2[...])
    c0_o[...] = c0
    v0_o[...] = v0
    a1_o[...] = _dot(c0, fl1[...]) + flb1[...]
    b1_o[...] = _dot(v0, fr1[...])
    a2_o[...] = _dot(v0, fl2[...]) + flb2[...]


def _tc_mid_body(t1, degc, c0, ffw, ffb, o1a, o1b, o1bias, o2w, o2b, fr2,
                 b2_o):
    agg = _dot(t1[...], ffw[...]) + degc[...][:, 0:1] * ffb[...]
    h = jnp.tanh(_dot(agg, o1a[...]) + _dot(c0[...], o1b[...]) + o1bias[...])
    c1 = _dot(h, o2w[...]) + o2b[...]
    b2_o[...] = _dot(c1, fr2[...])


def _tc_fin_body(t2, degv, v0, ffw, ffb, o1a, o1b, o1bias, o2w, o2b,
                 w1, b1, w2, out_o):
    agg = _dot(t2[...], ffw[...]) + degv[...][:, 0:1] * ffb[...]
    h = jnp.tanh(_dot(agg, o1a[...]) + _dot(v0[...], o1b[...]) + o1bias[...])
    v1 = _dot(h, o2w[...]) + o2b[...]
    out_o[...] = _dot(jnp.tanh(_dot(v1, w1[...]) + b1[...]), w2[...])


def _tc_call(body, ins, widths_in, outs_shapes):
    grid = (N_NODES // BM,)
    in_specs = []
    for x, w in zip(ins, widths_in):
        in_specs.append(_rows_spec(w) if w is not None else _full_spec(x.shape))
    return pl.pallas_call(
        body,
        grid=grid,
        in_specs=in_specs,
        out_specs=[_rows_spec(s[1]) for s in outs_shapes],
        out_shape=[jax.ShapeDtypeStruct(s, jnp.float32) for s in outs_shapes],
    )(*ins)


def _split_halves(x):
    return jnp.concatenate([x[:, :H], x[:, H:]], axis=0)


def kernel(constraint_features, edge_indices, edge_features, variable_features, params):
    p = params
    eidx = edge_indices.astype(jnp.int32)
    ew = edge_features[:, 0]
    vc, cv = p["vc"], p["cv"]

    def r1(b):
        return b.reshape(1, -1)

    c0, v0, a1, b1, a2 = _tc_call(
        _tc0_body,
        [constraint_features, variable_features,
         p["c_emb"]["W1"], r1(p["c_emb"]["b1"]), p["c_emb"]["W2"], r1(p["c_emb"]["b2"]),
         p["v_emb"]["W1"], r1(p["v_emb"]["b1"]), p["v_emb"]["W2"], r1(p["v_emb"]["b2"]),
         vc["fl_W"], r1(vc["fl_b"]), vc["fr_W"], cv["fl_W"], r1(cv["fl_b"])],
        [5, 17] + [None] * 13,
        [(N_NODES, D)] * 5,
    )

    degs = _sc_deg(eidx)

    t1 = _sc_conv(_split_halves(a1), _split_halves(b1),
                  eidx[0], eidx[1], ew, vc["fe_W"][0].reshape(2, H))
    t1_full = jnp.concatenate([t1[0], t1[1]], axis=1)

    (b2,) = _tc_call(
        _tc_mid_body,
        [t1_full, degs[0], c0,
         vc["ff_W"], r1(vc["ff_b"]), vc["o1_W"][:D], vc["o1_W"][D:],
         r1(vc["o1_b"]), vc["o2_W"], r1(vc["o2_b"]), cv["fr_W"]],
        [D, LANES, D] + [None] * 8,
        [(N_NODES, D)],
    )

    t2 = _sc_conv(_split_halves(a2), _split_halves(b2),
                  eidx[1], eidx[0], ew, cv["fe_W"][0].reshape(2, H))
    t2_full = jnp.concatenate([t2[0], t2[1]], axis=1)

    (out,) = _tc_call(
        _tc_fin_body,
        [t2_full, degs[1], v0,
         cv["ff_W"], r1(cv["ff_b"]), cv["o1_W"][:D], cv["o1_W"][D:],
         r1(cv["o1_b"]), cv["o2_W"], r1(cv["o2_b"]),
         p["out"]["W1"], r1(p["out"]["b1"]), p["out"]["W2"]],
        [D, LANES, D] + [None] * 10,
        [(N_NODES, 1)],
    )
    return out
